# Initial kernel scaffold; baseline (speedup 1.0000x reference)
import jax
import jax.numpy as jnp
from jax.experimental import pallas as pl


def kernel(x, edge_index, l1_lin_w, l1_lin_b, l1_att_l, l1_att_r, l1_fin_w, l1_fin_b, l2_lin_w, l2_lin_b, l2_att_l, l2_att_r, l2_fin_w, l2_fin_b, mp_w1, mp_b1, mp_w2, mp_b2):
    raise NotImplementedError("write your pallas kernel here")


# speedup vs baseline: 13.9902x; 13.9902x over previous
import functools

import jax
import jax.numpy as jnp
from jax import lax
from jax.experimental import pallas as pl
from jax.experimental.pallas import tpu as pltpu
from jax.experimental.pallas import tpu_sc as plsc

N = 10000
NP = 10240
E = 320000
H = 2
C = 128
SUB = 16
STRIPE = NP // SUB
ACCR = STRIPE + 8
ECH = 800
NSCAN = E // ECH
S1N = E // SUB // ECH
CCAP = 960
DRW = 64
BLK = 512
GRID = NP // BLK

_f32 = jnp.float32
_i32 = jnp.int32


def _pre1_body(x_ref, lw_ref, lb_ref, attl_ref, attr_ref,
               wh_ref, al_ref, ar_ref):
    xb = x_ref[...]
    lw = lw_ref[...]
    wb = lax.dot_general(xb, lw, (((1,), (1,)), ((), ())),
                         preferred_element_type=_f32) + lb_ref[...]
    w0 = wb[:, :C]
    w1 = wb[:, C:]
    wh_ref[0] = w0
    wh_ref[1] = w1
    al_ref[0, :] = (w0 * attl_ref[0][None, :]).sum(-1)
    al_ref[1, :] = (w1 * attl_ref[1][None, :]).sum(-1)
    ar_ref[0, :] = (w0 * attr_ref[0][None, :]).sum(-1)
    ar_ref[1, :] = (w1 * attr_ref[1][None, :]).sum(-1)


def _pre1(x, lin_w, lin_b, att_l, att_r):
    return pl.pallas_call(
        _pre1_body,
        grid=(GRID,),
        in_specs=[
            pl.BlockSpec((BLK, C), lambda i: (i, 0)),
            pl.BlockSpec((H * C, C), lambda i: (0, 0)),
            pl.BlockSpec((1, H * C), lambda i: (0, 0)),
            pl.BlockSpec((H, C), lambda i: (0, 0)),
            pl.BlockSpec((H, C), lambda i: (0, 0)),
        ],
        out_specs=[
            pl.BlockSpec((H, BLK, C), lambda i: (0, i, 0)),
            pl.BlockSpec((H, BLK), lambda i: (0, i)),
            pl.BlockSpec((H, BLK), lambda i: (0, i)),
        ],
        out_shape=[
            jax.ShapeDtypeStruct((H, NP, C), _f32),
            jax.ShapeDtypeStruct((H, NP), _f32),
            jax.ShapeDtypeStruct((H, NP), _f32),
        ],
    )(x, lin_w, lin_b.reshape(1, -1), att_l.reshape(H, C),
      att_r.reshape(H, C))


def _mid_body(x_ref, acc_ref, den_ref, fw_ref, fb_ref, lw_ref, lb_ref,
              attl_ref, attr_ref, h_ref, wh_ref, al_ref, ar_ref, *, din):
    a = acc_ref[...]
    d = den_ref[...]
    m0 = a[0] / (d[0][:, None] + 1e-16)
    m1 = a[1] / (d[1][:, None] + 1e-16)
    fw = fw_ref[...]
    dn = (((1,), (1,)), ((), ()))
    y = (lax.dot_general(x_ref[...], fw[:, :din], dn,
                         preferred_element_type=_f32)
         + lax.dot_general(m0, fw[:, din:din + C], dn,
                           preferred_element_type=_f32)
         + lax.dot_general(m1, fw[:, din + C:], dn,
                           preferred_element_type=_f32)
         + fb_ref[...])
    h = jnp.maximum(y, 0.0)
    h_ref[...] = h
    wb = lax.dot_general(h, lw_ref[...], dn,
                         preferred_element_type=_f32) + lb_ref[...]
    w0 = wb[:, :C]
    w1 = wb[:, C:]
    wh_ref[0] = w0
    wh_ref[1] = w1
    al_ref[0, :] = (w0 * attl_ref[0][None, :]).sum(-1)
    al_ref[1, :] = (w1 * attl_ref[1][None, :]).sum(-1)
    ar_ref[0, :] = (w0 * attr_ref[0][None, :]).sum(-1)
    ar_ref[1, :] = (w1 * attr_ref[1][None, :]).sum(-1)


def _mid(x, acc, den, fin_w, fin_b, l2_lin_w, l2_lin_b, l2_att_l, l2_att_r):
    din = x.shape[1]
    return pl.pallas_call(
        functools.partial(_mid_body, din=din),
        grid=(GRID,),
        in_specs=[
            pl.BlockSpec((BLK, din), lambda i: (i, 0)),
            pl.BlockSpec((H, BLK, C), lambda i: (0, i, 0)),
            pl.BlockSpec((H, BLK), lambda i: (0, i)),
            pl.BlockSpec((H * C, din + H * C), lambda i: (0, 0)),
            pl.BlockSpec((1, H * C), lambda i: (0, 0)),
            pl.BlockSpec((H * C, H * C), lambda i: (0, 0)),
            pl.BlockSpec((1, H * C), lambda i: (0, 0)),
            pl.BlockSpec((H, C), lambda i: (0, 0)),
            pl.BlockSpec((H, C), lambda i: (0, 0)),
        ],
        out_specs=[
            pl.BlockSpec((BLK, H * C), lambda i: (i, 0)),
            pl.BlockSpec((H, BLK, C), lambda i: (0, i, 0)),
            pl.BlockSpec((H, BLK), lambda i: (0, i)),
            pl.BlockSpec((H, BLK), lambda i: (0, i)),
        ],
        out_shape=[
            jax.ShapeDtypeStruct((NP, H * C), _f32),
            jax.ShapeDtypeStruct((H, NP, C), _f32),
            jax.ShapeDtypeStruct((H, NP), _f32),
            jax.ShapeDtypeStruct((H, NP), _f32),
        ],
    )(x, acc, den, fin_w, fin_b.reshape(1, -1), l2_lin_w,
      l2_lin_b.reshape(1, -1), l2_att_l.reshape(H, C),
      l2_att_r.reshape(H, C))


def _post_body(x_ref, acc_ref, den_ref, fw_ref, fb_ref, w1_ref, b1_ref,
               w2_ref, b2_ref, out_ref):
    a = acc_ref[...]
    d = den_ref[...]
    m0 = a[0] / (d[0][:, None] + 1e-16)
    m1 = a[1] / (d[1][:, None] + 1e-16)
    fw = fw_ref[...]
    dn = (((1,), (1,)), ((), ()))
    din = H * C
    y = (lax.dot_general(x_ref[...], fw[:, :din], dn,
                         preferred_element_type=_f32)
         + lax.dot_general(m0, fw[:, din:din + C], dn,
                           preferred_element_type=_f32)
         + lax.dot_general(m1, fw[:, din + C:], dn,
                           preferred_element_type=_f32)
         + fb_ref[...])
    h = jnp.maximum(y, 0.0)
    t = lax.dot_general(h, w1_ref[...], dn,
                        preferred_element_type=_f32) + b1_ref[...]
    t = lax.dot_general(t, w2_ref[...], dn,
                        preferred_element_type=_f32) + b2_ref[...]
    m = jnp.max(t, axis=1, keepdims=True)
    z = t - m
    out_ref[...] = z - jnp.log(jnp.sum(jnp.exp(z), axis=1, keepdims=True))


def _post(x, acc, den, fin_w, fin_b, mp_w1, mp_b1, mp_w2, mp_b2):
    din = H * C
    nclass = mp_w2.shape[0]
    return pl.pallas_call(
        _post_body,
        grid=(GRID,),
        in_specs=[
            pl.BlockSpec((BLK, din), lambda i: (i, 0)),
            pl.BlockSpec((H, BLK, C), lambda i: (0, i, 0)),
            pl.BlockSpec((H, BLK), lambda i: (0, i)),
            pl.BlockSpec((H * C, din + H * C), lambda i: (0, 0)),
            pl.BlockSpec((1, H * C), lambda i: (0, 0)),
            pl.BlockSpec((128, H * C), lambda i: (0, 0)),
            pl.BlockSpec((1, 128), lambda i: (0, 0)),
            pl.BlockSpec((nclass, 128), lambda i: (0, 0)),
            pl.BlockSpec((1, nclass), lambda i: (0, 0)),
        ],
        out_specs=pl.BlockSpec((BLK, nclass), lambda i: (i, 0)),
        out_shape=jax.ShapeDtypeStruct((NP, nclass), _f32),
    )(x, acc, den, fin_w, fin_b.reshape(1, -1), mp_w1, mp_b1.reshape(1, -1),
      mp_w2, mp_b2.reshape(1, -1))


def _sc_edge_body(w_hbm, al_hbm, ar_hbm, src_hbm, dst_hbm, zeros_hbm,
                  zeros_n_hbm, out_msg, out_den, al_v, ar_v, esrc_v,
                  edst_v, csrc_v, cdl_v, sdr_v, rows_v, acc_v, den_v,
                  gsem):
    c = lax.axis_index("c")
    t = lax.axis_index("s")
    base = t * STRIPE

    pltpu.sync_copy(al_hbm.at[c], al_v.at[pl.ds(0, NP)])
    pltpu.sync_copy(ar_hbm.at[c], ar_v.at[pl.ds(0, NP)])
    al_v[pl.ds(NP, 16)] = jnp.zeros((16,), _f32)
    ar_v[pl.ds(NP, 16)] = jnp.zeros((16,), _f32)
    pltpu.sync_copy(zeros_hbm.at[pl.ds(0, ACCR)], acc_v)
    pltpu.sync_copy(zeros_n_hbm.at[pl.ds(0, 656)], den_v)

    lane = lax.iota(_i32, 16)
    lane0 = lane == 0
    cols = [lane + cc * 16 for cc in range(C // 16)]
    w_h = w_hbm.at[c]

    def drain_body(dj, _):
        off2 = dj * DRW
        gat = pltpu.async_copy(w_h.at[csrc_v.at[pl.ds(off2, DRW)]], rows_v,
                               gsem)
        for g in range(DRW // 16):
            sidx = csrc_v[pl.ds(off2 + g * 16, 16)]
            didx = cdl_v[pl.ds(off2 + g * 16, 16)] + base
            a = plsc.load_gather(al_v, [sidx]) + plsc.load_gather(
                ar_v, [didx])
            sdr_v[pl.ds(g * 16, 16)] = jnp.exp(jnp.maximum(a, 0.2 * a))
        gat.wait()
        for e in range(DRW):
            dspl = plsc.load_gather(cdl_v, [jnp.full((16,), off2 + e, _i32)])
            sspl = plsc.load_gather(sdr_v, [jnp.full((16,), e, _i32)])
            for cc in range(C // 16):
                plsc.addupdate_scatter(
                    acc_v, [dspl, cols[cc]],
                    rows_v[e, pl.ds(cc * 16, 16)] * sspl)
            plsc.addupdate_scatter(den_v, [dspl], sspl, mask=lane0)
        return 0

    def scan_chunk(ci, cnt):
        off = ci * ECH
        pltpu.sync_copy(src_hbm.at[pl.ds(off, ECH)], esrc_v)
        pltpu.sync_copy(dst_hbm.at[pl.ds(off, ECH)], edst_v)
        for v in range(ECH // 16):
            dl = edst_v[pl.ds(v * 16, 16)] - base
            m = (dl >= 0) & (dl < STRIPE)
            plsc.store_compressed(csrc_v.at[pl.ds(cnt, 16)],
                                  esrc_v[pl.ds(v * 16, 16)], mask=m)
            plsc.store_compressed(cdl_v.at[pl.ds(cnt, 16)], dl, mask=m)
            cnt = cnt + jnp.max(plsc.all_reduce_population_count(m))
        k = cnt // DRW
        lax.fori_loop(0, k, drain_body, 0)
        for q in range(4):
            tmp_s = csrc_v[pl.ds(k * DRW + q * 16, 16)]
            tmp_d = cdl_v[pl.ds(k * DRW + q * 16, 16)]
            csrc_v[pl.ds(q * 16, 16)] = tmp_s
            cdl_v[pl.ds(q * 16, 16)] = tmp_d
        return cnt - k * DRW

    cnt = lax.fori_loop(0, NSCAN, scan_chunk, jnp.int32(0))
    ftrue = jnp.full((16,), True)
    for q in range(4):
        plsc.store_compressed(csrc_v.at[pl.ds(cnt + q * 16, 16)],
                              jnp.zeros((16,), _i32), mask=ftrue)
        plsc.store_compressed(cdl_v.at[pl.ds(cnt + q * 16, 16)],
                              jnp.full((16,), STRIPE, _i32), mask=ftrue)
    lax.fori_loop(0, (cnt + DRW - 1) // DRW, drain_body, 0)

    pltpu.sync_copy(acc_v.at[pl.ds(0, STRIPE)],
                    out_msg.at[c, pl.ds(base, STRIPE)])
    pltpu.sync_copy(den_v.at[pl.ds(0, STRIPE)],
                    out_den.at[c, pl.ds(base, STRIPE)])


def _sc_edge(w_heads, al, ar, src, dst, zeros_hbm, zeros_n):
    mesh = plsc.VectorSubcoreMesh(core_axis_name="c", subcore_axis_name="s")
    f = pl.kernel(
        _sc_edge_body,
        mesh=mesh,
        compiler_params=pltpu.CompilerParams(needs_layout_passes=False),
        out_type=[
            jax.ShapeDtypeStruct((H, NP, C), _f32),
            jax.ShapeDtypeStruct((H, NP), _f32),
        ],
        scratch_types=[
            pltpu.VMEM((NP + 16,), _f32),
            pltpu.VMEM((NP + 16,), _f32),
            pltpu.VMEM((ECH,), _i32),
            pltpu.VMEM((ECH,), _i32),
            pltpu.VMEM((CCAP,), _i32),
            pltpu.VMEM((CCAP,), _i32),
            pltpu.VMEM((DRW,), _f32),
            pltpu.VMEM((DRW, C), _f32),
            pltpu.VMEM((ACCR, C), _f32),
            pltpu.VMEM((656,), _f32),
            pltpu.SemaphoreType.DMA,
        ],
    )
    return f(w_heads, al, ar, src, dst, zeros_hbm, zeros_n)


def kernel(x, edge_index, l1_lin_w, l1_lin_b, l1_att_l, l1_att_r, l1_fin_w,
           l1_fin_b, l2_lin_w, l2_lin_b, l2_att_l, l2_att_r, l2_fin_w,
           l2_fin_b, mp_w1, mp_b1, mp_w2, mp_b2):
    src = edge_index[0].astype(_i32)
    dst = edge_index[1].astype(_i32)
    zeros_hbm = jnp.zeros((NP, C), _f32)
    zeros_n = jnp.zeros((NP,), _f32)
    xp = jnp.pad(x, ((0, NP - N), (0, 0)))

    wh, al, ar = _pre1(xp, l1_lin_w, l1_lin_b, l1_att_l, l1_att_r)
    acc1, den1 = _sc_edge(wh, al, ar, src, dst, zeros_hbm, zeros_n)
    h, wh2, al2, ar2 = _mid(xp, acc1, den1, l1_fin_w, l1_fin_b, l2_lin_w,
                            l2_lin_b, l2_att_l, l2_att_r)
    acc2, den2 = _sc_edge(wh2, al2, ar2, src, dst, zeros_hbm, zeros_n)
    out = _post(h, acc2, den2, l2_fin_w, l2_fin_b, mp_w1, mp_b1, mp_w2,
                mp_b2)
    return out[:N]
